# Initial kernel scaffold; baseline (speedup 1.0000x reference)
import jax
import jax.numpy as jnp
from jax.experimental import pallas as pl


def kernel(edge_index, num_nodes):
    raise NotImplementedError("write your pallas kernel here")


# speedup vs baseline: 355.8372x; 355.8372x over previous
import functools

import jax
import jax.numpy as jnp
from jax import lax
from jax.experimental import pallas as pl
from jax.experimental.pallas import tpu as pltpu
from jax.experimental.pallas import tpu_sc as plsc

N_NODES = 100000
N_PAD = 102400
SLICE = N_PAD // 16
E = 6400000
NW = 32
EDGES_PER_TILE = E // NW
CHUNK = 4000
N_CHUNKS = EDGES_PER_TILE // CHUNK
GROUPS = CHUNK // 16

_MESH = plsc.VectorSubcoreMesh(core_axis_name="c", subcore_axis_name="s")
_SC_PARAMS = pltpu.CompilerParams(needs_layout_passes=False)


@functools.partial(
    pl.kernel,
    out_type=jax.ShapeDtypeStruct((2, N_PAD), jnp.float32),
    mesh=_MESH,
    scratch_types=[
        pltpu.VMEM((CHUNK,), jnp.int32),
        pltpu.VMEM((CHUNK,), jnp.float32),
        pltpu.VMEM_SHARED((N_PAD,), jnp.float32),
    ],
    compiler_params=_SC_PARAMS,
)
def _hist_kernel(col_hbm, zeros_hbm, ones_hbm, out_hbm, idx_v, ones_v, hist_s):
    cid = lax.axis_index("c")
    sid = lax.axis_index("s")
    wid = cid * 16 + sid
    pltpu.sync_copy(zeros_hbm, hist_s.at[pl.ds(sid * SLICE, SLICE)])
    pltpu.sync_copy(ones_hbm, ones_v)
    plsc.subcore_barrier()

    def chunk_body(k, carry):
        base = wid * EDGES_PER_TILE + k * CHUNK
        pltpu.sync_copy(col_hbm.at[pl.ds(base, CHUNK)], idx_v)
        pltpu.sync_copy(ones_v, hist_s.at[idx_v], add=True)
        return carry

    lax.fori_loop(0, N_CHUNKS, chunk_body, 0)
    plsc.subcore_barrier()
    pltpu.sync_copy(
        hist_s.at[pl.ds(sid * SLICE, SLICE)],
        out_hbm.at[cid, pl.ds(sid * SLICE, SLICE)],
    )


def _rsqrt_body(h_ref, o_ref):
    deg = h_ref[0, :, :] + h_ref[1, :, :]
    o_ref[...] = jnp.where(deg > 0.0, lax.rsqrt(deg), 0.0)


def _deg_inv_sqrt(hist):
    return pl.pallas_call(
        _rsqrt_body,
        out_shape=jax.ShapeDtypeStruct((N_PAD // 128, 128), jnp.float32),
    )(hist.reshape(2, N_PAD // 128, 128))


@functools.partial(
    pl.kernel,
    out_type=jax.ShapeDtypeStruct((E,), jnp.float32),
    mesh=_MESH,
    scratch_types=[
        pltpu.VMEM((N_PAD,), jnp.float32),
        pltpu.VMEM((CHUNK,), jnp.int32),
        pltpu.VMEM((CHUNK,), jnp.int32),
        pltpu.VMEM((CHUNK,), jnp.float32),
    ],
    compiler_params=_SC_PARAMS,
)
def _norm_kernel(row_hbm, col_hbm, dis_hbm, out_hbm, tab_v, row_v, col_v, out_v):
    cid = lax.axis_index("c")
    sid = lax.axis_index("s")
    wid = cid * 16 + sid
    pltpu.sync_copy(dis_hbm, tab_v)

    def chunk_body(k, carry):
        base = wid * EDGES_PER_TILE + k * CHUNK
        pltpu.sync_copy(row_hbm.at[pl.ds(base, CHUNK)], row_v)
        pltpu.sync_copy(col_hbm.at[pl.ds(base, CHUNK)], col_v)

        def group_body(g, c2):
            r = row_v[pl.ds(g * 16, 16)]
            c = col_v[pl.ds(g * 16, 16)]
            a = plsc.load_gather(tab_v, [r])
            b = plsc.load_gather(tab_v, [c])
            out_v[pl.ds(g * 16, 16)] = a * b
            return c2

        lax.fori_loop(0, GROUPS, group_body, 0)
        pltpu.sync_copy(out_v, out_hbm.at[pl.ds(base, CHUNK)])
        return carry

    lax.fori_loop(0, N_CHUNKS, chunk_body, 0)


def kernel(edge_index, num_nodes):
    del num_nodes
    row = edge_index[0]
    col = edge_index[1]
    zeros = jnp.zeros((SLICE,), jnp.float32)
    ones = jnp.ones((CHUNK,), jnp.float32)
    hist = _hist_kernel(col, zeros, ones)
    dis = _deg_inv_sqrt(hist).reshape(N_PAD)
    return _norm_kernel(row, col, dis)


# speedup vs baseline: 398.4790x; 1.1198x over previous
import functools

import jax
import jax.numpy as jnp
from jax import lax
from jax.experimental import pallas as pl
from jax.experimental.pallas import tpu as pltpu
from jax.experimental.pallas import tpu_sc as plsc

N_NODES = 100000
N_PAD = 102400
E = 6400000
NW = 32
EDGES_PER_TILE = E // NW

HCHUNK = 8000
H_CHUNKS = EDGES_PER_TILE // HCHUNK
H_GROUPS = HCHUNK // 80

CHUNK = 4000
N_CHUNKS = EDGES_PER_TILE // CHUNK
G_ITERS = CHUNK // 80

_MESH = plsc.VectorSubcoreMesh(core_axis_name="c", subcore_axis_name="s")
_SC_PARAMS = pltpu.CompilerParams(needs_layout_passes=False)


@functools.partial(
    pl.kernel,
    out_type=jax.ShapeDtypeStruct((NW, N_PAD), jnp.float32),
    mesh=_MESH,
    scratch_types=[
        pltpu.VMEM((2 * HCHUNK,), jnp.int32),
        pltpu.VMEM((N_PAD,), jnp.float32),
        pltpu.SemaphoreType.DMA,
    ],
    compiler_params=_SC_PARAMS,
)
def _hist_kernel(col_hbm, out_hbm, idx_v, hist_v, sem_in):
    cid = lax.axis_index("c")
    sid = lax.axis_index("s")
    wid = cid * 16 + sid
    ebase = wid * EDGES_PER_TILE

    pltpu.async_copy(col_hbm.at[pl.ds(ebase, HCHUNK)], idx_v.at[pl.ds(0, HCHUNK)], sem_in)

    zeros16 = jnp.zeros((16,), jnp.float32)

    def z_body(i, carry):
        base = i * 128
        for u in range(8):
            hist_v[pl.ds(base + u * 16, 16)] = zeros16
        return carry

    lax.fori_loop(0, N_PAD // 128, z_body, 0)

    ones16 = jnp.ones((16,), jnp.float32)

    def chunk_body(k, carry):
        b = lax.rem(k, 2)
        boff = b * HCHUNK
        pltpu.make_async_copy(
            col_hbm.at[pl.ds(ebase + k * HCHUNK, HCHUNK)],
            idx_v.at[pl.ds(boff, HCHUNK)],
            sem_in,
        ).wait()

        @pl.when(k + 1 < H_CHUNKS)
        def _():
            pltpu.async_copy(
                col_hbm.at[pl.ds(ebase + (k + 1) * HCHUNK, HCHUNK)],
                idx_v.at[pl.ds((1 - b) * HCHUNK, HCHUNK)],
                sem_in,
            )

        def g_body(g, c2):
            base = boff + g * 80
            for u in range(5):
                x = idx_v[pl.ds(base + u * 16, 16)]
                plsc.addupdate_scatter(hist_v, [x], ones16)
            return c2

        lax.fori_loop(0, H_GROUPS, g_body, 0)
        return carry

    lax.fori_loop(0, H_CHUNKS, chunk_body, 0)
    pltpu.sync_copy(hist_v, out_hbm.at[wid])


def _reduce_body(h_ref, o_ref):
    deg = jnp.sum(h_ref[...], axis=0)
    o_ref[...] = jnp.where(deg > 0.0, lax.rsqrt(deg), 0.0)


def _deg_inv_sqrt(hist):
    return pl.pallas_call(
        _reduce_body,
        out_shape=jax.ShapeDtypeStruct((N_PAD // 128, 128), jnp.float32),
    )(hist.reshape(NW, N_PAD // 128, 128))


@functools.partial(
    pl.kernel,
    out_type=jax.ShapeDtypeStruct((E,), jnp.float32),
    mesh=_MESH,
    scratch_types=[
        pltpu.VMEM((N_PAD,), jnp.float32),
        pltpu.VMEM((2 * CHUNK,), jnp.int32),
        pltpu.VMEM((2 * CHUNK,), jnp.int32),
        pltpu.VMEM((2 * CHUNK,), jnp.float32),
        pltpu.SemaphoreType.DMA,
        pltpu.SemaphoreType.DMA,
        pltpu.SemaphoreType.DMA,
    ],
    compiler_params=_SC_PARAMS,
)
def _norm_kernel(row_hbm, col_hbm, dis_hbm, out_hbm,
                 tab_v, row_v, col_v, out_v, sem_r, sem_c, sem_o):
    cid = lax.axis_index("c")
    sid = lax.axis_index("s")
    wid = cid * 16 + sid
    ebase = wid * EDGES_PER_TILE

    pltpu.async_copy(row_hbm.at[pl.ds(ebase, CHUNK)], row_v.at[pl.ds(0, CHUNK)], sem_r)
    pltpu.async_copy(col_hbm.at[pl.ds(ebase, CHUNK)], col_v.at[pl.ds(0, CHUNK)], sem_c)
    pltpu.sync_copy(dis_hbm, tab_v)

    def chunk_body(k, carry):
        b = lax.rem(k, 2)
        boff = b * CHUNK
        base = ebase + k * CHUNK
        pltpu.make_async_copy(
            row_hbm.at[pl.ds(base, CHUNK)], row_v.at[pl.ds(boff, CHUNK)], sem_r
        ).wait()
        pltpu.make_async_copy(
            col_hbm.at[pl.ds(base, CHUNK)], col_v.at[pl.ds(boff, CHUNK)], sem_c
        ).wait()

        @pl.when(k + 1 < N_CHUNKS)
        def _():
            noff = (1 - b) * CHUNK
            nbase = base + CHUNK
            pltpu.async_copy(row_hbm.at[pl.ds(nbase, CHUNK)], row_v.at[pl.ds(noff, CHUNK)], sem_r)
            pltpu.async_copy(col_hbm.at[pl.ds(nbase, CHUNK)], col_v.at[pl.ds(noff, CHUNK)], sem_c)

        @pl.when(k >= 2)
        def _():
            pltpu.make_async_copy(
                out_v.at[pl.ds(boff, CHUNK)], out_hbm.at[pl.ds(base, CHUNK)], sem_o
            ).wait()

        def group_body(g, c2):
            goff = g * 80
            for u in range(5):
                off = boff + goff + u * 16
                r = row_v[pl.ds(off, 16)]
                c = col_v[pl.ds(off, 16)]
                a = plsc.load_gather(tab_v, [r])
                bb = plsc.load_gather(tab_v, [c])
                out_v[pl.ds(off, 16)] = a * bb
            return c2

        lax.fori_loop(0, G_ITERS, group_body, 0)
        pltpu.async_copy(out_v.at[pl.ds(boff, CHUNK)], out_hbm.at[pl.ds(base, CHUNK)], sem_o)
        return carry

    lax.fori_loop(0, N_CHUNKS, chunk_body, 0)
    last = ebase + (N_CHUNKS - 1) * CHUNK
    pltpu.make_async_copy(
        out_v.at[pl.ds(0, CHUNK)], out_hbm.at[pl.ds(last, CHUNK)], sem_o
    ).wait()
    pltpu.make_async_copy(
        out_v.at[pl.ds(0, CHUNK)], out_hbm.at[pl.ds(last, CHUNK)], sem_o
    ).wait()


def kernel(edge_index, num_nodes):
    del num_nodes
    row = edge_index[0]
    col = edge_index[1]
    hist = _hist_kernel(col)
    dis = _deg_inv_sqrt(hist).reshape(N_PAD)
    return _norm_kernel(row, col, dis)


# speedup vs baseline: 763.7420x; 1.9166x over previous
import functools

import jax
import jax.numpy as jnp
from jax import lax
from jax.experimental import pallas as pl
from jax.experimental.pallas import tpu as pltpu
from jax.experimental.pallas import tpu_sc as plsc

N_NODES = 100000
N_PAD = 102400
E = 6400000
NW = 32
EDGES_PER_TILE = E // NW

HCHUNK = 8000
H_CHUNKS = EDGES_PER_TILE // HCHUNK
H_GROUPS = HCHUNK // 16

CHUNK = 4000
N_CHUNKS = EDGES_PER_TILE // CHUNK
GROUPS = CHUNK // 16

_MESH = plsc.VectorSubcoreMesh(core_axis_name="c", subcore_axis_name="s")
_SC_PARAMS = pltpu.CompilerParams(needs_layout_passes=False)


@functools.partial(
    pl.kernel,
    out_type=jax.ShapeDtypeStruct((NW, N_PAD), jnp.float32),
    mesh=_MESH,
    scratch_types=[
        pltpu.VMEM((2 * HCHUNK,), jnp.int32),
        pltpu.VMEM((N_PAD,), jnp.float32),
        pltpu.SemaphoreType.DMA,
    ],
    compiler_params=_SC_PARAMS,
)
def _hist_kernel(edge_hbm, out_hbm, idx_v, hist_v, sem_in):
    cid = lax.axis_index("c")
    sid = lax.axis_index("s")
    wid = cid * 16 + sid
    ebase = wid * EDGES_PER_TILE

    pltpu.async_copy(
        edge_hbm.at[pl.ds(E + ebase, HCHUNK)], idx_v.at[pl.ds(0, HCHUNK)], sem_in
    )

    zeros16 = jnp.zeros((16,), jnp.float32)

    @functools.partial(plsc.parallel_loop, 0, N_PAD // 16, unroll=8)
    def _(i):
        hist_v[pl.ds(i * 16, 16)] = zeros16

    ones16 = jnp.ones((16,), jnp.float32)

    def chunk_body(k, carry):
        b = lax.rem(k, 2)
        boff = b * HCHUNK
        pltpu.make_async_copy(
            edge_hbm.at[pl.ds(E + ebase + k * HCHUNK, HCHUNK)],
            idx_v.at[pl.ds(boff, HCHUNK)],
            sem_in,
        ).wait()

        @pl.when(k + 1 < H_CHUNKS)
        def _():
            pltpu.async_copy(
                edge_hbm.at[pl.ds(E + ebase + (k + 1) * HCHUNK, HCHUNK)],
                idx_v.at[pl.ds((1 - b) * HCHUNK, HCHUNK)],
                sem_in,
            )

        @functools.partial(plsc.parallel_loop, 0, H_GROUPS, unroll=10)
        def _(g):
            x = idx_v[pl.ds(boff + g * 16, 16)]
            plsc.addupdate_scatter(hist_v, [x], ones16)

        return carry

    lax.fori_loop(0, H_CHUNKS, chunk_body, 0)
    pltpu.sync_copy(hist_v, out_hbm.at[wid])


def _reduce_body(h_ref, o_ref):
    deg = jnp.sum(h_ref[...], axis=0)
    o_ref[...] = jnp.where(deg > 0.0, lax.rsqrt(deg), 0.0)


def _deg_inv_sqrt(hist):
    return pl.pallas_call(
        _reduce_body,
        out_shape=jax.ShapeDtypeStruct((N_PAD // 128, 128), jnp.float32),
    )(hist.reshape(NW, N_PAD // 128, 128))


@functools.partial(
    pl.kernel,
    out_type=jax.ShapeDtypeStruct((E,), jnp.float32),
    mesh=_MESH,
    scratch_types=[
        pltpu.VMEM((N_PAD,), jnp.float32),
        pltpu.VMEM((2 * CHUNK,), jnp.int32),
        pltpu.VMEM((2 * CHUNK,), jnp.int32),
        pltpu.VMEM((2 * CHUNK,), jnp.float32),
        pltpu.SemaphoreType.DMA,
        pltpu.SemaphoreType.DMA,
        pltpu.SemaphoreType.DMA,
    ],
    compiler_params=_SC_PARAMS,
)
def _norm_kernel(edge_hbm, dis_hbm, out_hbm,
                 tab_v, row_v, col_v, out_v, sem_r, sem_c, sem_o):
    cid = lax.axis_index("c")
    sid = lax.axis_index("s")
    wid = cid * 16 + sid
    ebase = wid * EDGES_PER_TILE

    pltpu.async_copy(edge_hbm.at[pl.ds(ebase, CHUNK)], row_v.at[pl.ds(0, CHUNK)], sem_r)
    pltpu.async_copy(edge_hbm.at[pl.ds(E + ebase, CHUNK)], col_v.at[pl.ds(0, CHUNK)], sem_c)
    pltpu.sync_copy(dis_hbm, tab_v)

    def chunk_body(k, carry):
        b = lax.rem(k, 2)
        boff = b * CHUNK
        base = ebase + k * CHUNK
        pltpu.make_async_copy(
            edge_hbm.at[pl.ds(base, CHUNK)], row_v.at[pl.ds(boff, CHUNK)], sem_r
        ).wait()
        pltpu.make_async_copy(
            edge_hbm.at[pl.ds(E + base, CHUNK)], col_v.at[pl.ds(boff, CHUNK)], sem_c
        ).wait()

        @pl.when(k + 1 < N_CHUNKS)
        def _():
            noff = (1 - b) * CHUNK
            nbase = base + CHUNK
            pltpu.async_copy(edge_hbm.at[pl.ds(nbase, CHUNK)], row_v.at[pl.ds(noff, CHUNK)], sem_r)
            pltpu.async_copy(edge_hbm.at[pl.ds(E + nbase, CHUNK)], col_v.at[pl.ds(noff, CHUNK)], sem_c)

        @pl.when(k >= 2)
        def _():
            pltpu.make_async_copy(
                out_v.at[pl.ds(boff, CHUNK)], out_hbm.at[pl.ds(base, CHUNK)], sem_o
            ).wait()

        @functools.partial(plsc.parallel_loop, 0, GROUPS, unroll=10)
        def _(g):
            off = boff + g * 16
            r = row_v[pl.ds(off, 16)]
            c = col_v[pl.ds(off, 16)]
            a = plsc.load_gather(tab_v, [r])
            bb = plsc.load_gather(tab_v, [c])
            out_v[pl.ds(off, 16)] = a * bb

        pltpu.async_copy(out_v.at[pl.ds(boff, CHUNK)], out_hbm.at[pl.ds(base, CHUNK)], sem_o)
        return carry

    lax.fori_loop(0, N_CHUNKS, chunk_body, 0)
    last = ebase + (N_CHUNKS - 1) * CHUNK
    pltpu.make_async_copy(
        out_v.at[pl.ds(0, CHUNK)], out_hbm.at[pl.ds(last, CHUNK)], sem_o
    ).wait()
    pltpu.make_async_copy(
        out_v.at[pl.ds(0, CHUNK)], out_hbm.at[pl.ds(last, CHUNK)], sem_o
    ).wait()


def kernel(edge_index, num_nodes):
    del num_nodes
    edge_flat = edge_index.reshape(2 * E)
    hist = _hist_kernel(edge_flat)
    dis = _deg_inv_sqrt(hist).reshape(N_PAD)
    return _norm_kernel(edge_flat, dis)
